# baseline (speedup 1.0000x reference)
import functools

import jax
import jax.numpy as jnp
from jax import lax
from jax.experimental import pallas as pl
from jax.experimental.pallas import tpu as pltpu
from jax.experimental.pallas import tpu_sc as plsc

_ENT_SIZE = 100000
_EMB_DIM = 128
_BATCH = 16384

_NC = 2
_NS = 16
_NW = _NC * _NS
_BPW = _BATCH // _NW
_C = 16
_NCHUNK = _BPW // _C
_NGRP = _C // 16
_NBUF = 8


def _sc_body(head_hbm, rel_hbm, tail_hbm,
             ent_emb, ent_covar, rel_emb, rel_covar,
             out_hbm,
             hidx, ridx, tidx,
             *scratch):
    wid = lax.axis_index("s") * _NC + lax.axis_index("c")
    base = wid * _BPW

    lanes = lax.iota(jnp.int32, 16)

    pltpu.sync_copy(head_hbm.at[pl.ds(base, _BPW)], hidx)
    pltpu.sync_copy(rel_hbm.at[pl.ds(base, _BPW)], ridx)
    pltpu.sync_copy(tail_hbm.at[pl.ds(base, _BPW)], tidx)

    bufsets = tuple(tuple(scratch[6 * s:6 * s + 6]) for s in range(_NBUF))
    out_v = scratch[6 * _NBUF]
    sems = scratch[6 * _NBUF + 1:]

    def fire(s, off):
        hi = hidx.at[pl.ds(off, _C)]
        ri = ridx.at[pl.ds(off, _C)]
        ti = tidx.at[pl.ds(off, _C)]
        srcs = (ent_emb.at[hi], ent_covar.at[hi],
                ent_emb.at[ti], ent_covar.at[ti],
                rel_emb.at[ri], rel_covar.at[ri])
        return [pltpu.async_copy(src, buf, sems[s])
                for src, buf in zip(srcs, bufsets[s])]

    rot = lanes * 9
    zero = jnp.zeros((16,), jnp.float32)

    def compute(s, off):
        hm, hv, tm, tv, rm, rv = bufsets[s]

        def body(j, accs):
            col = (j + rot) & (_EMB_DIM - 1)
            new = []
            for g in range(_NGRP):
                rows = lanes + g * 16
                a = plsc.load_gather(hm, [rows, col])
                b = plsc.load_gather(hv, [rows, col])
                cm = plsc.load_gather(tm, [rows, col])
                cv = plsc.load_gather(tv, [rows, col])
                dm = plsc.load_gather(rm, [rows, col])
                dv = plsc.load_gather(rv, [rows, col])
                ev = cv + b
                d_ = dm - (cm - a)
                d2 = d_ * d_
                num = (ev + d2) * ev + (dv + d2) * dv
                new.append(accs[g] + num / (dv * ev))
            return tuple(new)

        accs = lax.fori_loop(0, _EMB_DIM, body, (zero,) * _NGRP)
        for g in range(_NGRP):
            out_v[pl.ds(off + g * 16, 16)] = (accs[g] - 2.0 * _EMB_DIM) * 0.25

    waiters = [
        [pltpu.make_async_copy(ent_emb.at[hidx.at[pl.ds(0, _C)]], buf, sems[s])
         for buf in bufsets[s]]
        for s in range(_NBUF)
    ]
    for s in range(_NBUF - 1):
        fire(s, s * _C)

    def ring(k, _):
        cbase = k * _NBUF
        for s in range(_NBUF):
            c = cbase + s
            off = c * _C

            @pl.when(c + _NBUF - 1 < _NCHUNK)
            def _():
                fire((s + _NBUF - 1) % _NBUF, off + (_NBUF - 1) * _C)

            for d in waiters[s]:
                d.wait()
            compute(s, off)
        return 0

    lax.fori_loop(0, _NCHUNK // _NBUF, ring, 0)

    pltpu.sync_copy(out_v, out_hbm.at[pl.ds(base, _BPW)])


_sc_kernel = functools.partial(
    pl.kernel,
    out_type=jax.ShapeDtypeStruct((_BATCH,), jnp.float32),
    mesh=plsc.VectorSubcoreMesh(core_axis_name="c", subcore_axis_name="s"),
    scratch_types=[
        pltpu.VMEM((_BPW,), jnp.int32),
        pltpu.VMEM((_BPW,), jnp.int32),
        pltpu.VMEM((_BPW,), jnp.int32),
    ] + [pltpu.VMEM((_C, _EMB_DIM), jnp.float32)] * (6 * _NBUF) + [
        pltpu.VMEM((_BPW,), jnp.float32),
    ] + [pltpu.SemaphoreType.DMA] * _NBUF,
    compiler_params=pltpu.CompilerParams(
        use_tc_tiling_on_sc=False, needs_layout_passes=False,
        skip_device_barrier=True),
)(_sc_body)


def kernel(in_triple, ent_emb, ent_covar, rel_emb, rel_covar):
    head = in_triple[:, 0]
    rel = in_triple[:, 1]
    tail = in_triple[:, 2]
    return _sc_kernel(head, rel, tail, ent_emb, ent_covar, rel_emb, rel_covar)


# speedup vs baseline: 1.0149x; 1.0149x over previous
import functools

import jax
import jax.numpy as jnp
from jax import lax
from jax.experimental import pallas as pl
from jax.experimental.pallas import tpu as pltpu
from jax.experimental.pallas import tpu_sc as plsc

_ENT_SIZE = 100000
_EMB_DIM = 128
_BATCH = 16384

_NC = 2
_NS = 16
_NW = _NC * _NS
_BPW = _BATCH // _NW
_C = 32
_NCHUNK = _BPW // _C
_NGRP = _C // 16
_NBUF = 4


def _sc_body(head_hbm, rel_hbm, tail_hbm,
             ent_emb, ent_covar, rel_emb, rel_covar,
             out_hbm,
             hidx, ridx, tidx,
             *scratch):
    wid = lax.axis_index("s") * _NC + lax.axis_index("c")
    base = wid * _BPW

    lanes = lax.iota(jnp.int32, 16)

    pltpu.sync_copy(head_hbm.at[pl.ds(base, _BPW)], hidx)
    pltpu.sync_copy(rel_hbm.at[pl.ds(base, _BPW)], ridx)
    pltpu.sync_copy(tail_hbm.at[pl.ds(base, _BPW)], tidx)

    bufsets = tuple(tuple(scratch[6 * s:6 * s + 6]) for s in range(_NBUF))
    out_v = scratch[6 * _NBUF]
    sems = scratch[6 * _NBUF + 1:]

    def fire(s, off):
        hi = hidx.at[pl.ds(off, _C)]
        ri = ridx.at[pl.ds(off, _C)]
        ti = tidx.at[pl.ds(off, _C)]
        srcs = (ent_emb.at[hi], ent_covar.at[hi],
                ent_emb.at[ti], ent_covar.at[ti],
                rel_emb.at[ri], rel_covar.at[ri])
        return [pltpu.async_copy(src, buf, sems[s])
                for src, buf in zip(srcs, bufsets[s])]

    rot = lanes * 9
    zero = jnp.zeros((16,), jnp.float32)

    def compute(s, off):
        hm, hv, tm, tv, rm, rv = bufsets[s]

        def body(j, accs):
            col = (j + rot) & (_EMB_DIM - 1)
            new = []
            for g in range(_NGRP):
                rows = lanes + g * 16
                a = plsc.load_gather(hm, [rows, col])
                b = plsc.load_gather(hv, [rows, col])
                cm = plsc.load_gather(tm, [rows, col])
                cv = plsc.load_gather(tv, [rows, col])
                dm = plsc.load_gather(rm, [rows, col])
                dv = plsc.load_gather(rv, [rows, col])
                ev = cv + b
                d_ = dm - (cm - a)
                d2 = d_ * d_
                num = (ev + d2) * ev + (dv + d2) * dv
                new.append(accs[g] + num / (dv * ev))
            return tuple(new)

        accs = lax.fori_loop(0, _EMB_DIM, body, (zero,) * _NGRP)
        for g in range(_NGRP):
            out_v[pl.ds(off + g * 16, 16)] = (accs[g] - 2.0 * _EMB_DIM) * 0.25

    waiters = [
        [pltpu.make_async_copy(ent_emb.at[hidx.at[pl.ds(0, _C)]], buf, sems[s])
         for buf in bufsets[s]]
        for s in range(_NBUF)
    ]
    for s in range(_NBUF - 1):
        fire(s, s * _C)

    def ring(k, _):
        cbase = k * _NBUF
        for s in range(_NBUF):
            c = cbase + s
            off = c * _C

            @pl.when(c + _NBUF - 1 < _NCHUNK)
            def _():
                fire((s + _NBUF - 1) % _NBUF, off + (_NBUF - 1) * _C)

            for d in waiters[s]:
                d.wait()
            compute(s, off)
        return 0

    lax.fori_loop(0, _NCHUNK // _NBUF, ring, 0)

    pltpu.sync_copy(out_v, out_hbm.at[pl.ds(base, _BPW)])


_sc_kernel = functools.partial(
    pl.kernel,
    out_type=jax.ShapeDtypeStruct((_BATCH,), jnp.float32),
    mesh=plsc.VectorSubcoreMesh(core_axis_name="c", subcore_axis_name="s"),
    scratch_types=[
        pltpu.VMEM((_BPW,), jnp.int32),
        pltpu.VMEM((_BPW,), jnp.int32),
        pltpu.VMEM((_BPW,), jnp.int32),
    ] + [pltpu.VMEM((_C, _EMB_DIM), jnp.float32)] * (6 * _NBUF) + [
        pltpu.VMEM((_BPW,), jnp.float32),
    ] + [pltpu.SemaphoreType.DMA] * _NBUF,
    compiler_params=pltpu.CompilerParams(
        use_tc_tiling_on_sc=False, needs_layout_passes=False,
        skip_device_barrier=True,
        disable_bounds_checks=True, disable_semaphore_checks=True),
)(_sc_body)


def kernel(in_triple, ent_emb, ent_covar, rel_emb, rel_covar):
    head = in_triple[:, 0]
    rel = in_triple[:, 1]
    tail = in_triple[:, 2]
    return _sc_kernel(head, rel, tail, ent_emb, ent_covar, rel_emb, rel_covar)


# speedup vs baseline: 1.0412x; 1.0259x over previous
import functools

import jax
import jax.numpy as jnp
from jax import lax
from jax.experimental import pallas as pl
from jax.experimental.pallas import tpu as pltpu
from jax.experimental.pallas import tpu_sc as plsc

_ENT_SIZE = 100000
_EMB_DIM = 128
_BATCH = 16384

_NC = 2
_NS = 16
_NW = _NC * _NS
_BPW = _BATCH // _NW
_C = 32
_NCHUNK = _BPW // _C
_NGRP = _C // 16
_NBUF = 4


def _sc_body(head_hbm, rel_hbm, tail_hbm,
             ent_emb, ent_covar, rel_emb, rel_covar,
             out_hbm,
             hidx, ridx, tidx,
             *scratch):
    wid = lax.axis_index("s") * _NC + lax.axis_index("c")
    base = wid * _BPW

    lanes = lax.iota(jnp.int32, 16)

    isem = scratch[-1]
    i1 = pltpu.async_copy(head_hbm.at[pl.ds(base, _BPW)], hidx, isem)
    i2 = pltpu.async_copy(rel_hbm.at[pl.ds(base, _BPW)], ridx, isem)
    i3 = pltpu.async_copy(tail_hbm.at[pl.ds(base, _BPW)], tidx, isem)
    i1.wait()
    i2.wait()
    i3.wait()

    bufsets = tuple(tuple(scratch[6 * s:6 * s + 6]) for s in range(_NBUF))
    out_v = scratch[6 * _NBUF]
    sems = scratch[6 * _NBUF + 1:6 * _NBUF + 1 + _NBUF]

    def fire(s, off):
        hi = hidx.at[pl.ds(off, _C)]
        ri = ridx.at[pl.ds(off, _C)]
        ti = tidx.at[pl.ds(off, _C)]
        srcs = (ent_emb.at[hi], ent_covar.at[hi],
                ent_emb.at[ti], ent_covar.at[ti],
                rel_emb.at[ri], rel_covar.at[ri])
        return [pltpu.async_copy(src, buf, sems[s])
                for src, buf in zip(srcs, bufsets[s])]

    rot = lanes * 9
    zero = jnp.zeros((16,), jnp.float32)

    def compute(s, off):
        hm, hv, tm, tv, rm, rv = bufsets[s]

        def body(j, accs):
            col = (j + rot) & (_EMB_DIM - 1)
            new = []
            for g in range(_NGRP):
                rows = lanes + g * 16
                a = plsc.load_gather(hm, [rows, col])
                b = plsc.load_gather(hv, [rows, col])
                cm = plsc.load_gather(tm, [rows, col])
                cv = plsc.load_gather(tv, [rows, col])
                dm = plsc.load_gather(rm, [rows, col])
                dv = plsc.load_gather(rv, [rows, col])
                ev = cv + b
                d_ = dm - (cm - a)
                d2 = d_ * d_
                num = (ev + d2) * ev + (dv + d2) * dv
                new.append(accs[g] + num / (dv * ev))
            return tuple(new)

        accs = lax.fori_loop(0, _EMB_DIM, body, (zero,) * _NGRP)
        for g in range(_NGRP):
            out_v[pl.ds(off + g * 16, 16)] = (accs[g] - 2.0 * _EMB_DIM) * 0.25

    waiters = [
        [pltpu.make_async_copy(ent_emb.at[hidx.at[pl.ds(0, _C)]], buf, sems[s])
         for buf in bufsets[s]]
        for s in range(_NBUF)
    ]
    for s in range(_NBUF - 1):
        fire(s, s * _C)

    def ring(k, _):
        cbase = k * _NBUF
        for s in range(_NBUF):
            c = cbase + s
            off = c * _C

            @pl.when(c + _NBUF - 1 < _NCHUNK)
            def _():
                fire((s + _NBUF - 1) % _NBUF, off + (_NBUF - 1) * _C)

            for d in waiters[s]:
                d.wait()
            compute(s, off)
        return 0

    lax.fori_loop(0, _NCHUNK // _NBUF, ring, 0)

    pltpu.sync_copy(out_v, out_hbm.at[pl.ds(base, _BPW)])


_sc_kernel = functools.partial(
    pl.kernel,
    out_type=jax.ShapeDtypeStruct((_BATCH,), jnp.float32),
    mesh=plsc.VectorSubcoreMesh(core_axis_name="c", subcore_axis_name="s"),
    scratch_types=[
        pltpu.VMEM((_BPW,), jnp.int32),
        pltpu.VMEM((_BPW,), jnp.int32),
        pltpu.VMEM((_BPW,), jnp.int32),
    ] + [pltpu.VMEM((_C, _EMB_DIM), jnp.float32)] * (6 * _NBUF) + [
        pltpu.VMEM((_BPW,), jnp.float32),
    ] + [pltpu.SemaphoreType.DMA] * (_NBUF + 1),
    compiler_params=pltpu.CompilerParams(
        use_tc_tiling_on_sc=False, needs_layout_passes=False,
        skip_device_barrier=True,
        disable_bounds_checks=True, disable_semaphore_checks=True),
)(_sc_body)


def kernel(in_triple, ent_emb, ent_covar, rel_emb, rel_covar):
    head = in_triple[:, 0]
    rel = in_triple[:, 1]
    tail = in_triple[:, 2]
    return _sc_kernel(head, rel, tail, ent_emb, ent_covar, rel_emb, rel_covar)
